# baseline (speedup 1.0000x reference)
import jax
import jax.numpy as jnp
from jax.experimental import pallas as pl


_RT = 1024


def _add_pos_kernel(x_ref, pos_ref, out_ref):
    out_ref[0, :, :] = x_ref[0, :, :] + pos_ref[...]


def kernel(x, pos_embedding):
    B, T, D = x.shape
    rt = _RT if T % _RT == 0 else T
    grid = (T // rt, B)
    return pl.pallas_call(
        _add_pos_kernel,
        grid=grid,
        in_specs=[
            pl.BlockSpec((1, rt, D), lambda i, b: (b, i, 0)),
            pl.BlockSpec((rt, D), lambda i, b: (i, 0)),
        ],
        out_specs=pl.BlockSpec((1, rt, D), lambda i, b: (b, i, 0)),
        out_shape=jax.ShapeDtypeStruct((B, T, D), x.dtype),
    )(x, pos_embedding)
